# baseline (speedup 1.0000x reference)
import jax
import jax.numpy as jnp
from jax import lax
from jax.experimental import pallas as pl
from jax.experimental.pallas import tpu as pltpu
from jax.experimental.pallas import tpu_sc as plsc

NC = 2
NS = 16
LANES = 16
CHUNK = 80
NBUF = 3


def _make_spmm(n, e, d):
    rows_per_tile = n // NS
    ept = e // (NC * NS)
    nchunk = ept // CHUNK
    fregs = d // LANES
    zrows = CHUNK
    nzcopy = rows_per_tile // zrows

    mesh = plsc.VectorSubcoreMesh(
        core_axis_name="c", subcore_axis_name="s", num_cores=NC,
        num_subcores=NS)

    scratch = [
        pltpu.VMEM((ept,), jnp.int32),
        pltpu.VMEM((NBUF, CHUNK), jnp.int32),
        pltpu.VMEM((NBUF, CHUNK), jnp.int32),
        pltpu.VMEM((NBUF, CHUNK), jnp.float32),
        pltpu.VMEM((NBUF, CHUNK, d), jnp.float32),
        pltpu.VMEM_SHARED((n, d), jnp.float32),
        [pltpu.SemaphoreType.DMA] * NBUF,
        [pltpu.SemaphoreType.DMA] * NBUF,
        [pltpu.SemaphoreType.DMA] * NBUF,
    ]

    def body(y_hbm, packed_hbm, w_hbm, out_hbm, packedv, colu, rowu, wvr,
             rowsb, acc, gsems, ssems, wsems):
        c = lax.axis_index("c")
        s = lax.axis_index("s")
        wid = c * NS + s
        r0 = s * rows_per_tile
        ebase = wid * ept

        pltpu.sync_copy(packed_hbm.at[pl.ds(ebase, ept)], packedv)

        zeros = jnp.zeros((LANES,), jnp.float32)

        def zero_row(i, _):
            for f in range(fregs):
                rowsb[0, i, pl.ds(f * LANES, LANES)] = zeros
            return 0

        lax.fori_loop(0, zrows, zero_row, 0)

        def zero_copy(i, _):
            pltpu.sync_copy(rowsb.at[0],
                            acc.at[pl.ds(r0 + i * zrows, zrows)])
            return 0

        lax.fori_loop(0, nzcopy, zero_copy, 0)
        plsc.subcore_barrier()

        def unpack_idx(g, b):
            for grp in range(CHUNK // LANES):
                pv = packedv[pl.ds(g * CHUNK + grp * LANES, LANES)]
                sl = pl.ds(grp * LANES, LANES)
                colu[b, sl] = pv & 0xFFFF
                rowu[b, sl] = lax.shift_right_logical(pv, 16)

        def start_w(g, b):
            pltpu.async_copy(w_hbm.at[pl.ds(ebase + g * CHUNK, CHUNK)],
                             wvr.at[b], wsems[b])

        def wait_w(g, b):
            pltpu.make_async_copy(
                w_hbm.at[pl.ds(ebase + g * CHUNK, CHUNK)], wvr.at[b],
                wsems[b]).wait()

        def start_g(g, b):
            pltpu.async_copy(y_hbm.at[colu.at[b]], rowsb.at[b], gsems[b])

        def wait_g(g, b):
            pltpu.make_async_copy(y_hbm.at[colu.at[b]], rowsb.at[b],
                                  gsems[b]).wait()

        def start_s(g, b):
            pltpu.async_copy(rowsb.at[b], acc.at[rowu.at[b]], ssems[b],
                             add=True)

        def wait_s(g, b):
            pltpu.make_async_copy(rowsb.at[b], acc.at[rowu.at[b]],
                                  ssems[b]).wait()

        def scale(g, b):
            for grp in range(CHUNK // LANES):
                wvec = wvr[b, pl.ds(grp * LANES, LANES)]
                for j in range(LANES):
                    ei = grp * LANES + j
                    wsplat = wvec.at[jnp.full((LANES,), j, jnp.int32)].get(
                        mode="promise_in_bounds")
                    for f in range(fregs):
                        sl = pl.ds(f * LANES, LANES)
                        rowsb[b, ei, sl] = rowsb[b, ei, sl] * wsplat

        def pipe_iter(g, b, first, has_next):
            wait_w(g, b)
            wait_g(g, b)
            scale(g, b)
            start_s(g, b)
            if not first:
                wait_s(g - 1, (b - 1) % NBUF)
            if has_next:
                bn = (b + 2) % NBUF
                unpack_idx(g + 2, bn)
                start_w(g + 2, bn)
                start_g(g + 2, bn)

        for i in range(2):
            unpack_idx(i, i)
            start_w(i, i)
            start_g(i, i)

        pipe_iter(0, 0, True, True)
        pipe_iter(1, 1, False, True)

        ntrip = (nchunk - 2) // NBUF

        def trip_body(t, _):
            g0 = 2 + t * NBUF
            for k in range(NBUF):
                g = g0 + k
                b = (2 + k) % NBUF
                wait_w(g, b)
                wait_g(g, b)
                scale(g, b)
                start_s(g, b)
                wait_s(g - 1, (b - 1) % NBUF)

                @pl.when(g + 2 < nchunk)
                def _():
                    bn = (b + 2) % NBUF
                    unpack_idx(g + 2, bn)
                    start_w(g + 2, bn)
                    start_g(g + 2, bn)

            return 0

        lax.fori_loop(0, ntrip, trip_body, 0)

        for g in range(2 + ntrip * NBUF, nchunk):
            pipe_iter(g, g % NBUF, False, g + 2 < nchunk)

        wait_s(nchunk - 1, (nchunk - 1) % NBUF)
        plsc.subcore_barrier()

        pltpu.sync_copy(acc.at[pl.ds(r0, rows_per_tile)],
                        out_hbm.at[c].at[pl.ds(r0, rows_per_tile)])

    return pl.kernel(
        body,
        out_type=jax.ShapeDtypeStruct((NC, n, d), jnp.float32),
        mesh=mesh,
        scratch_types=scratch,
        name="cheby_spmm",
    )


def _combine(p, prev, alpha):
    _, n, d = p.shape
    bn = 1024
    grid = (n // bn,)
    has_prev = prev is not None

    def body(*refs):
        if has_prev:
            p_ref, prev_ref, o_ref = refs
        else:
            p_ref, o_ref = refs
            prev_ref = None
        acc = (p_ref[0] + p_ref[1]) * alpha
        if has_prev:
            acc = acc - prev_ref[...]
        o_ref[...] = acc

    in_specs = [pl.BlockSpec((2, bn, d), lambda i: (0, i, 0))]
    args = [p]
    if has_prev:
        in_specs.append(pl.BlockSpec((bn, d), lambda i: (i, 0)))
        args.append(prev)
    return pl.pallas_call(
        body,
        grid=grid,
        in_specs=in_specs,
        out_specs=pl.BlockSpec((bn, d), lambda i: (i, 0)),
        out_shape=jax.ShapeDtypeStruct((n, d), jnp.float32),
    )(*args)


def _combine_mm1(t1p, x, W0, W1, bias):
    _, n, d = t1p.shape
    bn = 1024
    grid = (n // bn,)

    def body(p_ref, x_ref, w0_ref, w1_ref, b_ref, t1_ref, o_ref):
        t1 = p_ref[0] + p_ref[1]
        t1_ref[...] = t1
        acc = jnp.dot(x_ref[...], w0_ref[...],
                      preferred_element_type=jnp.float32)
        acc += jnp.dot(t1, w1_ref[...], preferred_element_type=jnp.float32)
        o_ref[...] = acc + b_ref[...]

    r_spec = pl.BlockSpec((bn, d), lambda i: (i, 0))
    w_spec = pl.BlockSpec((d, d), lambda i: (0, 0))
    return pl.pallas_call(
        body,
        grid=grid,
        in_specs=[pl.BlockSpec((2, bn, d), lambda i: (0, i, 0)), r_spec,
                  w_spec, w_spec, pl.BlockSpec((1, d), lambda i: (0, 0))],
        out_specs=[r_spec, r_spec],
        out_shape=[jax.ShapeDtypeStruct((n, d), jnp.float32),
                   jax.ShapeDtypeStruct((n, d), jnp.float32)],
    )(t1p, x, W0, W1, bias.reshape(1, d))


def _combine_mm2(t2p, x, pB, W2):
    _, n, d = t2p.shape
    bn = 1024
    grid = (n // bn,)

    def body(p_ref, x_ref, pb_ref, w2_ref, t2_ref, o_ref):
        t2 = (p_ref[0] + p_ref[1]) * 2.0 - x_ref[...]
        t2_ref[...] = t2
        o_ref[...] = pb_ref[...] + jnp.dot(
            t2, w2_ref[...], preferred_element_type=jnp.float32)

    r_spec = pl.BlockSpec((bn, d), lambda i: (i, 0))
    return pl.pallas_call(
        body,
        grid=grid,
        in_specs=[pl.BlockSpec((2, bn, d), lambda i: (0, i, 0)), r_spec,
                  r_spec, pl.BlockSpec((d, d), lambda i: (0, 0))],
        out_specs=[r_spec, r_spec],
        out_shape=[jax.ShapeDtypeStruct((n, d), jnp.float32),
                   jax.ShapeDtypeStruct((n, d), jnp.float32)],
    )(t2p, x, pB, W2)


def _final_fused(base, t3p, t1, W3):
    n, d = base.shape
    bn = 1024
    grid = (n // bn,)

    def body(p_ref, t1_ref, w_ref, base_ref, o_ref):
        t3 = (p_ref[0] + p_ref[1]) * 2.0 - t1_ref[...]
        o_ref[...] = base_ref[...] + jnp.dot(
            t3, w_ref[...], preferred_element_type=jnp.float32)

    r_spec = pl.BlockSpec((bn, d), lambda i: (i, 0))
    return pl.pallas_call(
        body,
        grid=grid,
        in_specs=[pl.BlockSpec((2, bn, d), lambda i: (0, i, 0)), r_spec,
                  pl.BlockSpec((d, d), lambda i: (0, 0)), r_spec],
        out_specs=r_spec,
        out_shape=jax.ShapeDtypeStruct((n, d), jnp.float32),
    )(t3p, t1, W3, base)


def kernel(x, edge_index, edge_weight, W0, W1, W2, W3, bias):
    n, d = x.shape
    e = edge_index.shape[1]
    row = edge_index[0]
    col = edge_index[1]

    npad = ((n + NS * 8 - 1) // (NS * 8)) * (NS * 8)
    npad = ((npad + 1023) // 1024) * 1024
    x_pad = jnp.pad(x, ((0, npad - n), (0, 0)))

    spmm = _make_spmm(npad, e, d)

    packed = (row << 16) | col

    t1p = spmm(x_pad, packed, edge_weight)
    t1, pB = _combine_mm1(t1p, x_pad, W0, W1, bias)
    t2p = spmm(t1, packed, edge_weight)
    t2, pC = _combine_mm2(t2p, x_pad, pB, W2)
    t3p = spmm(t2, packed, edge_weight)

    out = _final_fused(pC, t3p, t1, W3)
    return out[:n]


# speedup vs baseline: 1.0176x; 1.0176x over previous
import jax
import jax.numpy as jnp
from jax import lax
from jax.experimental import pallas as pl
from jax.experimental.pallas import tpu as pltpu
from jax.experimental.pallas import tpu_sc as plsc

NC = 2
NS = 16
LANES = 16
CHUNK = 80
NBUF = 3


def _make_spmm(n, e, d):
    rows_per_tile = n // NS
    ept = e // (NC * NS)
    nchunk = ept // CHUNK
    fregs = d // LANES
    zrows = CHUNK
    nzcopy = rows_per_tile // zrows

    mesh = plsc.VectorSubcoreMesh(
        core_axis_name="c", subcore_axis_name="s", num_cores=NC,
        num_subcores=NS)

    scratch = [
        pltpu.VMEM((ept,), jnp.int32),
        pltpu.VMEM((NBUF, CHUNK), jnp.int32),
        pltpu.VMEM((NBUF, CHUNK), jnp.int32),
        pltpu.VMEM((NBUF, CHUNK), jnp.float32),
        pltpu.VMEM((NBUF, CHUNK, d), jnp.float32),
        pltpu.VMEM_SHARED((n, d), jnp.float32),
        [pltpu.SemaphoreType.DMA] * NBUF,
        [pltpu.SemaphoreType.DMA] * NBUF,
        [pltpu.SemaphoreType.DMA] * NBUF,
    ]

    def body(y_hbm, packed_hbm, w_hbm, out_hbm, packedv, colu, rowu, wvr,
             rowsb, acc, gsems, ssems, wsems):
        c = lax.axis_index("c")
        s = lax.axis_index("s")
        wid = c * NS + s
        r0 = s * rows_per_tile
        ebase = wid * ept

        pltpu.sync_copy(packed_hbm.at[pl.ds(ebase, ept)], packedv)

        zeros = jnp.zeros((LANES,), jnp.float32)

        def zero_row(i, _):
            for f in range(fregs):
                rowsb[0, i, pl.ds(f * LANES, LANES)] = zeros
            return 0

        lax.fori_loop(0, zrows, zero_row, 0)

        def zero_copy(i, _):
            pltpu.sync_copy(rowsb.at[0],
                            acc.at[pl.ds(r0 + i * zrows, zrows)])
            return 0

        lax.fori_loop(0, nzcopy, zero_copy, 0)
        plsc.subcore_barrier()

        def unpack_idx(g, b):
            for grp in range(CHUNK // LANES):
                pv = packedv[pl.ds(g * CHUNK + grp * LANES, LANES)]
                sl = pl.ds(grp * LANES, LANES)
                colu[b, sl] = pv & 0xFFFF
                rowu[b, sl] = lax.shift_right_logical(pv, 16)

        def start_w(g, b):
            pltpu.async_copy(w_hbm.at[pl.ds(ebase + g * CHUNK, CHUNK)],
                             wvr.at[b], wsems[b])

        def wait_w(g, b):
            pltpu.make_async_copy(
                w_hbm.at[pl.ds(ebase + g * CHUNK, CHUNK)], wvr.at[b],
                wsems[b]).wait()

        def start_g(g, b):
            pltpu.async_copy(y_hbm.at[colu.at[b]], rowsb.at[b], gsems[b])

        def wait_g(g, b):
            pltpu.make_async_copy(y_hbm.at[colu.at[b]], rowsb.at[b],
                                  gsems[b]).wait()

        def start_s(g, b):
            pltpu.async_copy(rowsb.at[b], acc.at[rowu.at[b]], ssems[b],
                             add=True)

        def wait_s(g, b):
            pltpu.make_async_copy(rowsb.at[b], acc.at[rowu.at[b]],
                                  ssems[b]).wait()

        def scale(g, b):
            for grp in range(CHUNK // LANES):
                wvec = wvr[b, pl.ds(grp * LANES, LANES)]
                for j in range(LANES):
                    ei = grp * LANES + j
                    wsplat = wvec.at[jnp.full((LANES,), j, jnp.int32)].get(
                        mode="promise_in_bounds")
                    for f in range(fregs):
                        sl = pl.ds(f * LANES, LANES)
                        rowsb[b, ei, sl] = rowsb[b, ei, sl] * wsplat

        def pipe_iter(g, b, first, has_next):
            wait_w(g, b)
            wait_g(g, b)
            scale(g, b)
            start_s(g, b)
            if not first:
                wait_s(g - 1, (b - 1) % NBUF)
            if has_next:
                bn = (b + 2) % NBUF
                unpack_idx(g + 2, bn)
                start_w(g + 2, bn)
                start_g(g + 2, bn)

        for i in range(2):
            unpack_idx(i, i)
            start_w(i, i)
            start_g(i, i)

        pipe_iter(0, 0, True, True)
        pipe_iter(1, 1, False, True)

        ntrip = (nchunk - 2) // NBUF

        def trip_body(t, _):
            g0 = 2 + t * NBUF
            for k in range(NBUF):
                g = g0 + k
                b = (2 + k) % NBUF
                wait_w(g, b)
                wait_g(g, b)
                scale(g, b)
                start_s(g, b)
                wait_s(g - 1, (b - 1) % NBUF)

                @pl.when(g + 2 < nchunk)
                def _():
                    bn = (b + 2) % NBUF
                    unpack_idx(g + 2, bn)
                    start_w(g + 2, bn)
                    start_g(g + 2, bn)

            return 0

        lax.fori_loop(0, ntrip, trip_body, 0)

        for g in range(2 + ntrip * NBUF, nchunk):
            pipe_iter(g, g % NBUF, False, g + 2 < nchunk)

        wait_s(nchunk - 1, (nchunk - 1) % NBUF)
        plsc.subcore_barrier()

        pltpu.sync_copy(acc.at[pl.ds(r0, rows_per_tile)],
                        out_hbm.at[c].at[pl.ds(r0, rows_per_tile)])

    return pl.kernel(
        body,
        out_type=jax.ShapeDtypeStruct((NC, n, d), jnp.float32),
        mesh=mesh,
        scratch_types=scratch,
        name="cheby_spmm",
    )


def _combine_mm1(t1p, x, W0, W1, bias):
    _, n, d = t1p.shape
    bn = 1024
    grid = (n // bn,)

    def body(p_ref, x_ref, w0_ref, w1_ref, b_ref, t1_ref, o_ref):
        t1 = p_ref[0] + p_ref[1]
        t1_ref[...] = t1
        acc = jnp.dot(x_ref[...], w0_ref[...],
                      preferred_element_type=jnp.float32)
        acc += jnp.dot(t1, w1_ref[...], preferred_element_type=jnp.float32)
        o_ref[...] = acc + b_ref[...]

    r_spec = pl.BlockSpec((bn, d), lambda i: (i, 0))
    w_spec = pl.BlockSpec((d, d), lambda i: (0, 0))
    return pl.pallas_call(
        body,
        grid=grid,
        in_specs=[pl.BlockSpec((2, bn, d), lambda i: (0, i, 0)), r_spec,
                  w_spec, w_spec, pl.BlockSpec((1, d), lambda i: (0, 0))],
        out_specs=[r_spec, r_spec],
        out_shape=[jax.ShapeDtypeStruct((n, d), jnp.float32),
                   jax.ShapeDtypeStruct((n, d), jnp.float32)],
    )(t1p, x, W0, W1, bias.reshape(1, d))


def _combine_mm2(t2p, x, pB, W2):
    _, n, d = t2p.shape
    bn = 1024
    grid = (n // bn,)

    def body(p_ref, x_ref, pb_ref, w2_ref, t2_ref, o_ref):
        t2 = (p_ref[0] + p_ref[1]) * 2.0 - x_ref[...]
        t2_ref[...] = t2
        o_ref[...] = pb_ref[...] + jnp.dot(
            t2, w2_ref[...], preferred_element_type=jnp.float32)

    r_spec = pl.BlockSpec((bn, d), lambda i: (i, 0))
    return pl.pallas_call(
        body,
        grid=grid,
        in_specs=[pl.BlockSpec((2, bn, d), lambda i: (0, i, 0)), r_spec,
                  r_spec, pl.BlockSpec((d, d), lambda i: (0, 0))],
        out_specs=[r_spec, r_spec],
        out_shape=[jax.ShapeDtypeStruct((n, d), jnp.float32),
                   jax.ShapeDtypeStruct((n, d), jnp.float32)],
    )(t2p, x, pB, W2)


def _final_fused(base, t3p, t1, W3):
    n, d = base.shape
    bn = 1024
    grid = (n // bn,)

    def body(p_ref, t1_ref, w_ref, base_ref, o_ref):
        t3 = (p_ref[0] + p_ref[1]) * 2.0 - t1_ref[...]
        o_ref[...] = base_ref[...] + jnp.dot(
            t3, w_ref[...], preferred_element_type=jnp.float32)

    r_spec = pl.BlockSpec((bn, d), lambda i: (i, 0))
    return pl.pallas_call(
        body,
        grid=grid,
        in_specs=[pl.BlockSpec((2, bn, d), lambda i: (0, i, 0)), r_spec,
                  pl.BlockSpec((d, d), lambda i: (0, 0)), r_spec],
        out_specs=r_spec,
        out_shape=jax.ShapeDtypeStruct((n, d), jnp.float32),
    )(t3p, t1, W3, base)


def kernel(x, edge_index, edge_weight, W0, W1, W2, W3, bias):
    n, d = x.shape
    e = edge_index.shape[1]
    row = edge_index[0]
    col = edge_index[1]

    npad = ((n + NS * 8 - 1) // (NS * 8)) * (NS * 8)
    npad = ((npad + 1023) // 1024) * 1024
    x_pad = jnp.pad(x, ((0, npad - n), (0, 0)))

    spmm = _make_spmm(npad, e, d)

    packed = (row << 16) | col

    t1p = spmm(x_pad, packed, edge_weight)
    t1, pB = _combine_mm1(t1p, x_pad, W0, W1, bias)
    t2p = spmm(t1, packed, edge_weight)
    t2, pC = _combine_mm2(t2p, x_pad, pB, W2)
    t3p = spmm(t2, packed, edge_weight)

    out = _final_fused(pC, t3p, t1, W3)
    return out[:n]
